# Initial kernel scaffold; baseline (speedup 1.0000x reference)
import jax
import jax.numpy as jnp
from jax.experimental import pallas as pl


def kernel(feature, W1, b1, W2, b2, W3, b3):
    raise NotImplementedError("write your pallas kernel here")


# speedup vs baseline: 15.5840x; 15.5840x over previous
import jax
import jax.numpy as jnp
from jax.experimental import pallas as pl

_B = 16384
_D = 128
_K = 38
_ROWS = 512


def _body(feat_ref, w1_ref, b1_ref, w2_ref, b2_ref, w3_ref, b3_ref, out_ref):
    x = feat_ref[:]
    h = jnp.maximum(
        jnp.dot(x, w1_ref[:], preferred_element_type=jnp.float32) + b1_ref[:], 0.0
    )
    h = jnp.dot(h, w2_ref[:], preferred_element_type=jnp.float32) + b2_ref[:]
    t = jnp.dot(h, w3_ref[:], preferred_element_type=jnp.float32) + b3_ref[:]
    x2 = 1.0 / (1.0 + jnp.exp(-t))

    xi = jax.lax.bitcast_convert_type(x2, jnp.int32)
    prefix = jnp.zeros((x.shape[0], 1), dtype=jnp.int32)
    kk = jnp.full((x.shape[0], 1), _K - 1, dtype=jnp.int32)
    for b in range(29, -1, -1):
        hi_mask = jnp.int32(-(1 << b))
        match = (xi & hi_mask) == prefix
        c0 = jnp.sum(match.astype(jnp.int32), axis=1, keepdims=True)
        go1 = kk >= c0
        prefix = jnp.where(go1, prefix | jnp.int32(1 << b), prefix)
        kk = jnp.where(go1, kk - c0, kk)
    out_ref[:] = jnp.where(xi > prefix, x2, 0.0)


@jax.jit
def kernel(feature, W1, b1, W2, b2, W3, b3):
    a1 = W1.T
    a2 = W2.T
    a3 = W3.T
    bb1 = b1.reshape(1, _D)
    bb2 = b2.reshape(1, _D)
    bb3 = b3.reshape(1, _D)
    grid = _B // _ROWS
    row_spec = pl.BlockSpec((_ROWS, _D), lambda i: (i, 0))
    w_spec = pl.BlockSpec((_D, _D), lambda i: (0, 0))
    b_spec = pl.BlockSpec((1, _D), lambda i: (0, 0))
    return pl.pallas_call(
        _body,
        grid=(grid,),
        in_specs=[row_spec, w_spec, b_spec, w_spec, b_spec, w_spec, b_spec],
        out_specs=row_spec,
        out_shape=jax.ShapeDtypeStruct((_B, _D), jnp.float32),
    )(feature, a1, bb1, a2, bb2, a3, bb3)
